# baseline (speedup 1.0000x reference)
import functools

import jax
import jax.numpy as jnp
from jax import lax
from jax.experimental import pallas as pl
from jax.experimental.pallas import tpu as pltpu
from jax.experimental.pallas import tpu_sc as plsc

_B, _S, _EMB = 4096, 200, 128
_N = _B * _S
_NW = 32
_PER_W = _N // _NW
_CROWS = 2 * _S
_CHUNKS = _PER_W // _CROWS
_ISLOTS = 4


def _sc_embed(seq_flat, tok_table, pos_table):
    mesh = plsc.VectorSubcoreMesh(core_axis_name="c", subcore_axis_name="s")

    @functools.partial(
        pl.kernel,
        out_type=jax.ShapeDtypeStruct((_N, _EMB), jnp.float32),
        mesh=mesh,
        scratch_types=[
            pltpu.VMEM((_ISLOTS * _CROWS,), jnp.int32),
            pltpu.VMEM((_S, _EMB), jnp.float32),
            pltpu.VMEM((_CROWS, _EMB), jnp.float32),
            pltpu.VMEM((_CROWS, _EMB), jnp.float32),
            pltpu.SemaphoreType.DMA,
            pltpu.SemaphoreType.DMA,
            pltpu.SemaphoreType.DMA,
        ],
    )
    def k(seq_hbm, tok_hbm, pos_hbm, out_hbm, idxb, pos_v, buf0, buf1,
          sem_i, sem_g, sem_o):
        wid = lax.axis_index("s") * 2 + lax.axis_index("c")
        base = wid * _PER_W
        pltpu.sync_copy(pos_hbm.at[pl.ds(0, _S)], pos_v)
        bufs = (buf0, buf1)

        def idx_desc(c):
            return pltpu.make_async_copy(
                seq_hbm.at[pl.ds(base + c * _CROWS, _CROWS)],
                idxb.at[pl.ds((c % _ISLOTS) * _CROWS, _CROWS)], sem_i)

        def gather_descs(c, buf):
            sbase = (c % _ISLOTS) * _CROWS
            ds = []
            for off, n in ((0, 128), (128, 128), (256, 128), (384, 16)):
                ds.append(pltpu.make_async_copy(
                    tok_hbm.at[idxb.at[pl.ds(sbase + off, n)]],
                    buf.at[pl.ds(off, n)], sem_g))
            return ds

        def out_desc(c, buf, half):
            return pltpu.make_async_copy(
                buf.at[pl.ds(half * _S, _S)],
                out_hbm.at[pl.ds(base + c * _CROWS + half * _S, _S)], sem_o)

        def gather_start(c, buf):
            for d in gather_descs(c, buf):
                d.start()

        def gather_wait(c, buf):
            for d in gather_descs(c, buf):
                d.wait()

        def add_half(buf, half):
            def row_body(r, carry):
                for kk in range(_EMB // 16):
                    sl = pl.ds(kk * 16, 16)
                    plsc.addupdate(buf.at[r + half * _S, sl], pos_v[r, sl])
                return carry

            lax.fori_loop(0, _S, row_body, 0)

        for c0 in range(3):
            idx_desc(c0).start()
        idx_desc(0).wait()
        gather_start(0, bufs[0])

        def pair_body(t, carry):
            for b in range(2):
                c = 2 * t + b
                buf = bufs[b]
                other = bufs[1 - b]
                gather_wait(c, buf)

                @pl.when(c <= _CHUNKS - 2)
                def _prefetch():
                    idx_desc(c + 1).wait()

                    @pl.when(c >= 1)
                    def _free_buf():
                        out_desc(c - 1, other, 0).wait()
                        out_desc(c - 1, other, 1).wait()

                    gather_start(c + 1, other)

                @pl.when(c + 3 <= _CHUNKS - 1)
                def _stage_idx():
                    idx_desc(c + 3).start()

                add_half(buf, 0)
                out_desc(c, buf, 0).start()
                add_half(buf, 1)
                out_desc(c, buf, 1).start()
            return carry

        lax.fori_loop(0, _CHUNKS // 2, pair_body, 0)
        for c in (_CHUNKS - 2, _CHUNKS - 1):
            for half in (0, 1):
                out_desc(c, bufs[c % 2], half).wait()

    return k(seq_flat, tok_table, pos_table)


def kernel(seq, tok_table, pos_table):
    out = _sc_embed(seq.reshape(-1), tok_table, pos_table)
    return out.reshape(_B, _S, _EMB)


# speedup vs baseline: 1.0037x; 1.0037x over previous
import functools

import jax
import jax.numpy as jnp
from jax import lax
from jax.experimental import pallas as pl
from jax.experimental.pallas import tpu as pltpu
from jax.experimental.pallas import tpu_sc as plsc

_B, _S, _EMB = 4096, 200, 128
_N = _B * _S
_NW = 32
_PER_W = _N // _NW
_CROWS = 2 * _S
_CHUNKS = _PER_W // _CROWS
_ISLOTS = 4


def _sc_embed(seq_flat, tok_table, pos_table):
    mesh = plsc.VectorSubcoreMesh(core_axis_name="c", subcore_axis_name="s")

    @functools.partial(
        pl.kernel,
        out_type=jax.ShapeDtypeStruct((_N, _EMB), jnp.float32),
        mesh=mesh,
        scratch_types=[
            pltpu.VMEM((_ISLOTS * _CROWS,), jnp.int32),
            pltpu.VMEM((_S, _EMB), jnp.float32),
            pltpu.VMEM((_CROWS, _EMB), jnp.float32),
            pltpu.VMEM((_CROWS, _EMB), jnp.float32),
            pltpu.SemaphoreType.DMA,
            pltpu.SemaphoreType.DMA,
            pltpu.SemaphoreType.DMA,
        ],
    )
    def k(seq_hbm, tok_hbm, pos_hbm, out_hbm, idxb, pos_v, buf0, buf1,
          sem_i, sem_g, sem_o):
        wid = lax.axis_index("s") * 2 + lax.axis_index("c")
        base = wid * _PER_W
        pltpu.sync_copy(pos_hbm.at[pl.ds(0, _S)], pos_v)
        bufs = (buf0, buf1)

        def idx_desc(c):
            return pltpu.make_async_copy(
                seq_hbm.at[pl.ds(base + c * _CROWS, _CROWS)],
                idxb.at[pl.ds((c % _ISLOTS) * _CROWS, _CROWS)], sem_i)

        def gather_descs(c, buf):
            sbase = (c % _ISLOTS) * _CROWS
            ds = []
            for off, n in ((0, 128), (128, 128), (256, 128), (384, 16)):
                ds.append(pltpu.make_async_copy(
                    tok_hbm.at[idxb.at[pl.ds(sbase + off, n)]],
                    buf.at[pl.ds(off, n)], sem_g))
            return ds

        def out_desc(c, buf):
            return pltpu.make_async_copy(
                buf, out_hbm.at[pl.ds(base + c * _CROWS, _CROWS)], sem_o)

        def gather_start(c, buf):
            for d in gather_descs(c, buf):
                d.start()

        def gather_wait(c, buf):
            for d in gather_descs(c, buf):
                d.wait()

        def add_pos(buf):
            def row_body(r, carry):
                for kk in range(_EMB // 16):
                    sl = pl.ds(kk * 16, 16)
                    pv = pos_v[r, sl]
                    plsc.addupdate(buf.at[r, sl], pv)
                    plsc.addupdate(buf.at[r + _S, sl], pv)
                return carry

            lax.fori_loop(0, _S, row_body, 0)

        for c0 in range(3):
            idx_desc(c0).start()
        idx_desc(0).wait()
        gather_start(0, bufs[0])

        def pair_body(t, carry):
            for b in range(2):
                c = 2 * t + b
                buf = bufs[b]
                other = bufs[1 - b]
                gather_wait(c, buf)

                @pl.when(c <= _CHUNKS - 2)
                def _prefetch():
                    idx_desc(c + 1).wait()

                    @pl.when(c >= 1)
                    def _free_buf():
                        out_desc(c - 1, other).wait()

                    gather_start(c + 1, other)

                @pl.when(c + 3 <= _CHUNKS - 1)
                def _stage_idx():
                    idx_desc(c + 3).start()

                add_pos(buf)
                out_desc(c, buf).start()
            return carry

        lax.fori_loop(0, _CHUNKS // 2, pair_body, 0)
        out_desc(_CHUNKS - 2, bufs[0]).wait()
        out_desc(_CHUNKS - 1, bufs[1]).wait()

    return k(seq_flat, tok_table, pos_table)


def kernel(seq, tok_table, pos_table):
    out = _sc_embed(seq.reshape(-1), tok_table, pos_table)
    return out.reshape(_B, _S, _EMB)
